# baseline (speedup 1.0000x reference)
import functools

import jax
import jax.numpy as jnp
from jax import lax
from jax.experimental import pallas as pl
from jax.experimental.pallas import tpu as pltpu
from jax.experimental.pallas import tpu_sc as plsc

f32 = jnp.float32
bf16 = jnp.bfloat16

_N = 10000
_E = 160000
_D = 256
_DA = 288
_NP = 10240
_CH = 80
_NC, _NS = 2, 16
_EPT = _E // _NS
_NG = _EPT // _CH
_RPT = _NP // _NS


def _sc_aggregate(xb, ei):
  mesh = plsc.VectorSubcoreMesh(core_axis_name="c", subcore_axis_name="s")

  @functools.partial(
      pl.kernel,
      out_type=jax.ShapeDtypeStruct((_NC, _NP, _DA), bf16),
      mesh=mesh,
      scratch_types=[
          pltpu.VMEM((4, _CH), jnp.int32),
          pltpu.VMEM((4, _CH), jnp.int32),
          pltpu.VMEM((3, _CH, _DA), bf16),
          pltpu.VMEM_SHARED((_NP, _DA), bf16),
          pltpu.SemaphoreType.DMA((4,)),
          pltpu.SemaphoreType.DMA((3,)),
          pltpu.SemaphoreType.DMA((3,)),
      ],
      compiler_params=pltpu.CompilerParams(use_tc_tiling_on_sc=False),
  )
  def body(x_hbm, ei_hbm, out_hbm, sidx, didx, rows3, acc,
           sem_i, sem_r, sem_w):
    cid = lax.axis_index("c")
    sid = lax.axis_index("s")
    nb = (_NG + 1) // 2 - cid

    def idx_start(gg):
      g = 2 * gg + cid
      pltpu.async_copy(ei_hbm.at[0, sid, g], sidx.at[gg % 4],
                       sem_i.at[gg % 4])
      pltpu.async_copy(ei_hbm.at[1, sid, g], didx.at[gg % 4],
                       sem_i.at[gg % 4])

    def idx_wait(gg):
      g = 2 * gg + cid
      pltpu.make_async_copy(ei_hbm.at[0, sid, g], sidx.at[gg % 4],
                            sem_i.at[gg % 4]).wait()
      pltpu.make_async_copy(ei_hbm.at[1, sid, g], didx.at[gg % 4],
                            sem_i.at[gg % 4]).wait()

    def gather_start(gg):
      pltpu.async_copy(x_hbm.at[sidx.at[gg % 4]], rows3.at[gg % 3],
                       sem_r.at[gg % 3])

    def gather_wait(gg):
      pltpu.make_async_copy(x_hbm.at[sidx.at[gg % 4]], rows3.at[gg % 3],
                            sem_r.at[gg % 3]).wait()

    def scatter_start(gg):
      pltpu.async_copy(rows3.at[gg % 3], acc.at[didx.at[gg % 4]],
                       sem_w.at[gg % 3], add=True)

    def scatter_wait(gg):
      pltpu.make_async_copy(rows3.at[gg % 3], acc.at[didx.at[gg % 4]],
                            sem_w.at[gg % 3]).wait()

    idx_start(0)
    idx_start(1)

    zero32 = jnp.zeros((32,), bf16)
    rows0 = rows3.at[0]

    def zrows(k, c):
      rows0[k // (_DA // 32), pl.ds((k % (_DA // 32)) * 32, 32)] = zero32
      return c

    lax.fori_loop(0, _CH * (_DA // 32), zrows, 0)

    def zacc(k, c):
      pltpu.sync_copy(rows0, acc.at[pl.ds(sid * _RPT + k * _CH, _CH)])
      return c

    lax.fori_loop(0, _RPT // _CH, zacc, 0)
    plsc.subcore_barrier()

    idx_wait(0)
    gather_start(0)

    def step(gg, c):
      gather_wait(gg)

      @pl.when(gg >= 2)
      def _():
        scatter_wait(gg - 2)

      @pl.when(gg + 2 < nb)
      def _():
        idx_start(gg + 2)

      @pl.when(gg + 1 < nb)
      def _():
        idx_wait(gg + 1)
        gather_start(gg + 1)

      scatter_start(gg)
      return c

    lax.fori_loop(0, nb, step, 0)
    scatter_wait(nb - 2)
    scatter_wait(nb - 1)
    plsc.subcore_barrier()

    pltpu.sync_copy(acc.at[pl.ds(sid * _RPT, _RPT)],
                    out_hbm.at[cid, pl.ds(sid * _RPT, _RPT)])

  return body(xb, ei)


def _tc_self(x, W_r, b_l):
  blk = 1000
  grid = (_N // blk,)

  def body(x_ref, wr_ref, b_ref, o_ref):
    xb = x_ref[...]
    o_ref[...] = (jnp.dot(xb, wr_ref[...], preferred_element_type=f32)
                  + xb + b_ref[...])

  return pl.pallas_call(
      body,
      grid=grid,
      in_specs=[
          pl.BlockSpec((blk, _D), lambda g: (g, 0)),
          pl.BlockSpec((_D, _D), lambda g: (0, 0)),
          pl.BlockSpec((1, _D), lambda g: (0, 0)),
      ],
      out_specs=pl.BlockSpec((blk, _D), lambda g: (g, 0)),
      out_shape=jax.ShapeDtypeStruct((_N, _D), f32),
  )(x, W_r, b_l.reshape(1, _D))


def _tc_dense(h, out01, W_l):
  blk = 1000
  grid = (_N // blk,)

  def body(a0_ref, a1_ref, h_ref, wl_ref, o_ref):
    ab = a0_ref[0].astype(f32) + a1_ref[0].astype(f32)
    cnt = ab[:, _D:_D + 1]
    inv = 1.0 / jnp.maximum(cnt, 1.0)
    acc = jnp.dot(ab[:, :_D] * inv, wl_ref[...], preferred_element_type=f32)
    o_ref[...] = acc + h_ref[...]

  return pl.pallas_call(
      body,
      grid=grid,
      in_specs=[
          pl.BlockSpec((1, blk, _DA), lambda g: (0, g, 0)),
          pl.BlockSpec((1, blk, _DA), lambda g: (1, g, 0)),
          pl.BlockSpec((blk, _D), lambda g: (g, 0)),
          pl.BlockSpec((_D, _D), lambda g: (0, 0)),
      ],
      out_specs=pl.BlockSpec((blk, _D), lambda g: (g, 0)),
      out_shape=jax.ShapeDtypeStruct((_N, _D), f32),
  )(out01, out01, h, W_l)


def kernel(x, edge_index, W_l, b_l, W_r):
  xb = jnp.concatenate(
      [x.astype(bf16), jnp.ones((_N, _DA - _D), bf16)], axis=1)
  ei = edge_index.reshape(2, _NS, _NG, _CH)
  out01 = _sc_aggregate(xb, ei)
  h = _tc_self(x, W_r, b_l)
  return _tc_dense(h, out01, W_l)


# speedup vs baseline: 1.3281x; 1.3281x over previous
import functools

import jax
import jax.numpy as jnp
from jax import lax
from jax.experimental import pallas as pl
from jax.experimental.pallas import tpu as pltpu
from jax.experimental.pallas import tpu_sc as plsc

f32 = jnp.float32
bf16 = jnp.bfloat16

_N = 10000
_E = 160000
_D = 256
_NP = 10240
_CH = 80
_NC, _NS = 2, 16
_EPT = _E // _NS
_NG = _EPT // _CH
_RPT = _NP // _NS


def _sc_aggregate(xb, ei):
  mesh = plsc.VectorSubcoreMesh(core_axis_name="c", subcore_axis_name="s")

  @functools.partial(
      pl.kernel,
      out_type=[jax.ShapeDtypeStruct((_NC, _NP, _D), bf16),
                jax.ShapeDtypeStruct((_NC, _NP, 16), f32)],
      mesh=mesh,
      scratch_types=[
          pltpu.VMEM((4, _CH), jnp.int32),
          pltpu.VMEM((4, _CH), jnp.int32),
          pltpu.VMEM((3, _CH, _D), bf16),
          pltpu.VMEM((_CH, 16), f32),
          pltpu.VMEM_SHARED((_NP, _D), bf16),
          pltpu.VMEM_SHARED((_NP, 16), f32),
          pltpu.SemaphoreType.DMA((4,)),
          pltpu.SemaphoreType.DMA((3,)),
          pltpu.SemaphoreType.DMA((3,)),
      ],
      compiler_params=pltpu.CompilerParams(use_tc_tiling_on_sc=False),
  )
  def body(x_hbm, ei_hbm, out_hbm, outc_hbm, sidx, didx, rows3, ones,
           acc, cnt, sem_i, sem_r, sem_w):
    cid = lax.axis_index("c")
    sid = lax.axis_index("s")
    nb = (_NG + 1) // 2 - cid

    def idx_start(gg):
      g = 2 * gg + cid
      pltpu.async_copy(ei_hbm.at[0, sid, g], sidx.at[gg % 4],
                       sem_i.at[gg % 4])
      pltpu.async_copy(ei_hbm.at[1, sid, g], didx.at[gg % 4],
                       sem_i.at[gg % 4])

    def idx_wait(gg):
      g = 2 * gg + cid
      pltpu.make_async_copy(ei_hbm.at[0, sid, g], sidx.at[gg % 4],
                            sem_i.at[gg % 4]).wait()
      pltpu.make_async_copy(ei_hbm.at[1, sid, g], didx.at[gg % 4],
                            sem_i.at[gg % 4]).wait()

    def gather_start(gg):
      pltpu.async_copy(x_hbm.at[sidx.at[gg % 4]], rows3.at[gg % 3],
                       sem_r.at[gg % 3])

    def gather_wait(gg):
      pltpu.make_async_copy(x_hbm.at[sidx.at[gg % 4]], rows3.at[gg % 3],
                            sem_r.at[gg % 3]).wait()

    def scatter_start(gg):
      pltpu.async_copy(rows3.at[gg % 3], acc.at[didx.at[gg % 4]],
                       sem_w.at[gg % 3], add=True)
      pltpu.async_copy(ones, cnt.at[didx.at[gg % 4]], sem_w.at[gg % 3],
                       add=True)

    def scatter_wait(gg):
      pltpu.make_async_copy(rows3.at[gg % 3], acc.at[didx.at[gg % 4]],
                            sem_w.at[gg % 3]).wait()
      pltpu.make_async_copy(ones, cnt.at[didx.at[gg % 4]],
                            sem_w.at[gg % 3]).wait()

    idx_start(0)
    idx_start(1)

    zero = jnp.zeros((16,), f32)
    zero32 = jnp.zeros((32,), bf16)
    rows0 = rows3.at[0]

    def zrows(k, c):
      rows0[k // (_D // 32), pl.ds((k % (_D // 32)) * 32, 32)] = zero32
      return c

    lax.fori_loop(0, _CH * (_D // 32), zrows, 0)

    def zones(k, c):
      ones[k, pl.ds(0, 16)] = zero
      return c

    lax.fori_loop(0, _CH, zones, 0)

    def zacc(k, c):
      pltpu.sync_copy(rows0, acc.at[pl.ds(sid * _RPT + k * _CH, _CH)])
      return c

    lax.fori_loop(0, _RPT // _CH, zacc, 0)

    def zcnt(k, c):
      pltpu.sync_copy(ones, cnt.at[pl.ds(sid * _RPT + k * _CH, _CH)])
      return c

    lax.fori_loop(0, _RPT // _CH, zcnt, 0)

    one = jnp.ones((16,), f32)

    def fones(k, c):
      ones[k, pl.ds(0, 16)] = one
      return c

    lax.fori_loop(0, _CH, fones, 0)
    plsc.subcore_barrier()

    idx_wait(0)
    gather_start(0)

    def step(gg, c):
      gather_wait(gg)

      @pl.when(gg >= 2)
      def _():
        scatter_wait(gg - 2)

      @pl.when(gg + 2 < nb)
      def _():
        idx_start(gg + 2)

      @pl.when(gg + 1 < nb)
      def _():
        idx_wait(gg + 1)
        gather_start(gg + 1)

      scatter_start(gg)
      return c

    lax.fori_loop(0, nb, step, 0)
    scatter_wait(nb - 2)
    scatter_wait(nb - 1)
    plsc.subcore_barrier()

    pltpu.sync_copy(acc.at[pl.ds(sid * _RPT, _RPT)],
                    out_hbm.at[cid, pl.ds(sid * _RPT, _RPT)])
    pltpu.sync_copy(cnt.at[pl.ds(sid * _RPT, _RPT)],
                    outc_hbm.at[cid, pl.ds(sid * _RPT, _RPT)])

  return body(xb, ei)


def _tc_self(x, W_r, b_l):
  blk = 1000
  grid = (_N // blk,)

  def body(x_ref, wr_ref, b_ref, o_ref):
    xb = x_ref[...]
    o_ref[...] = (jnp.dot(xb, wr_ref[...], preferred_element_type=f32)
                  + xb + b_ref[...])

  return pl.pallas_call(
      body,
      grid=grid,
      in_specs=[
          pl.BlockSpec((blk, _D), lambda g: (g, 0)),
          pl.BlockSpec((_D, _D), lambda g: (0, 0)),
          pl.BlockSpec((1, _D), lambda g: (0, 0)),
      ],
      out_specs=pl.BlockSpec((blk, _D), lambda g: (g, 0)),
      out_shape=jax.ShapeDtypeStruct((_N, _D), f32),
  )(x, W_r, b_l.reshape(1, _D))


def _tc_dense(h, out01, outc, W_l):
  blk = 1000
  grid = (_N // blk,)

  def body(a0_ref, a1_ref, c0_ref, c1_ref, h_ref, wl_ref, o_ref):
    agg = a0_ref[0].astype(f32) + a1_ref[0].astype(f32)
    cnt = c0_ref[0][:, 0:1] + c1_ref[0][:, 0:1]
    inv = 1.0 / jnp.maximum(cnt, 1.0)
    acc = jnp.dot(agg * inv, wl_ref[...], preferred_element_type=f32)
    o_ref[...] = acc + h_ref[...]

  return pl.pallas_call(
      body,
      grid=grid,
      in_specs=[
          pl.BlockSpec((1, blk, _D), lambda g: (0, g, 0)),
          pl.BlockSpec((1, blk, _D), lambda g: (1, g, 0)),
          pl.BlockSpec((1, blk, 16), lambda g: (0, g, 0)),
          pl.BlockSpec((1, blk, 16), lambda g: (1, g, 0)),
          pl.BlockSpec((blk, _D), lambda g: (g, 0)),
          pl.BlockSpec((_D, _D), lambda g: (0, 0)),
      ],
      out_specs=pl.BlockSpec((blk, _D), lambda g: (g, 0)),
      out_shape=jax.ShapeDtypeStruct((_N, _D), f32),
  )(out01, out01, outc, outc, h, W_l)


def kernel(x, edge_index, W_l, b_l, W_r):
  xb = x.astype(bf16)
  ei = edge_index.reshape(2, _NS, _NG, _CH)
  out01, outc = _sc_aggregate(xb, ei)
  h = _tc_self(x, W_r, b_l)
  return _tc_dense(h, out01, outc, W_l)
